# baseline (speedup 1.0000x reference)
import functools

import jax
import jax.numpy as jnp
from jax import lax
from jax.experimental import pallas as pl
from jax.experimental.pallas import tpu as pltpu
from jax.experimental.pallas import tpu_sc as plsc

_N = 50000
_E = 800000
_F_IN = 128
_F = 64
_FE = 16
_G = 64

_NC = 2
_NS = 16
_NW = _NC * _NS

_EP = 819200
_NP = 51200

_NBLK = 2000
_EBLK = 4096
_UNBLK = 2048
_GCH = 800
_SCH = 6400

_f32 = jnp.float32


def _prep_body(x_ref, lf_ref, we_ref, be_ref, wi_ref, wj_ref,
               hi_ref, hj_ref):
    h = (jnp.dot(x_ref[...], we_ref[...], preferred_element_type=_f32)
         + be_ref[...] + lf_ref[...])
    hi_ref[...] = jnp.dot(h, wi_ref[...], preferred_element_type=_f32)
    hj_ref[...] = jnp.dot(h, wj_ref[...], preferred_element_type=_f32)


def _prep(x, lower_f, W_emb, b_emb, W_i, W_j):
    return pl.pallas_call(
        _prep_body,
        grid=(_N // _NBLK,),
        in_specs=[
            pl.BlockSpec((_NBLK, _F_IN), lambda i: (i, 0)),
            pl.BlockSpec((_NBLK, _F), lambda i: (i, 0)),
            pl.BlockSpec((_F_IN, _F), lambda i: (0, 0)),
            pl.BlockSpec((1, _F), lambda i: (0, 0)),
            pl.BlockSpec((_F, 2 * _F), lambda i: (0, 0)),
            pl.BlockSpec((_F, 2 * _F), lambda i: (0, 0)),
        ],
        out_specs=[
            pl.BlockSpec((_NBLK, 2 * _F), lambda i: (i, 0)),
            pl.BlockSpec((_NBLK, 2 * _F), lambda i: (i, 0)),
        ],
        out_shape=[
            jax.ShapeDtypeStruct((_N, 2 * _F), _f32),
            jax.ShapeDtypeStruct((_N, 2 * _F), _f32),
        ],
    )(x, lower_f, W_emb, b_emb.reshape(1, _F), W_i, W_j)


def _edge_mm_body(xi_ref, xj_ref, ea_ref, we_ref, b_ref, t_ref, st_ref):
    i = pl.program_id(0)
    t = (xi_ref[...] + xj_ref[...] + b_ref[...]
         + jnp.dot(ea_ref[...], we_ref[...], preferred_element_type=_f32))
    t_ref[...] = t

    @pl.when(i == 0)
    def _():
        st_ref[...] = jnp.zeros_like(st_ref)

    rows = lax.broadcasted_iota(jnp.int32, (_EBLK, 1), 0) + i * _EBLK
    tm = jnp.where(rows < _E, t, 0.0)
    st_ref[0:1, :] += jnp.sum(tm, axis=0, keepdims=True)
    st_ref[1:2, :] += jnp.sum(tm * tm, axis=0, keepdims=True)


def _edge_mm(xi, xj, ea, W_e, b_full):
    return pl.pallas_call(
        _edge_mm_body,
        grid=(_EP // _EBLK,),
        in_specs=[
            pl.BlockSpec((_EBLK, 2 * _F), lambda i: (i, 0)),
            pl.BlockSpec((_EBLK, 2 * _F), lambda i: (i, 0)),
            pl.BlockSpec((_EBLK, _FE), lambda i: (i, 0)),
            pl.BlockSpec((_FE, 2 * _F), lambda i: (0, 0)),
            pl.BlockSpec((1, 2 * _F), lambda i: (0, 0)),
        ],
        out_specs=[
            pl.BlockSpec((_EBLK, 2 * _F), lambda i: (i, 0)),
            pl.BlockSpec((2, 2 * _F), lambda i: (0, 0)),
        ],
        out_shape=[
            jax.ShapeDtypeStruct((_EP, 2 * _F), _f32),
            jax.ShapeDtypeStruct((2, 2 * _F), _f32),
        ],
    )(xi, xj, ea, W_e, b_full.reshape(1, 2 * _F))


_ESTEPS = _EP // _EBLK


def _msg_body(t_ref, st_ref, g_ref, b_ref, o_ref, mt_ref):
    i = pl.program_id(0)
    r = pl.program_id(1)
    rows = lax.broadcasted_iota(jnp.int32, (_EBLK, 1), 0) + i * _EBLK

    @pl.when(r == 0)
    def _():
        st = st_ref[...]
        mu = st[0:1, :] * (1.0 / _E)
        var = st[1:2, :] * (1.0 / _E) - mu * mu
        tn = (t_ref[...] - mu) * lax.rsqrt(var + 1e-5) * g_ref[...] + b_ref[...]
        filt = jax.nn.sigmoid(tn[:, :_F])
        core = jax.nn.softplus(tn[:, _F:])
        msg = jnp.where(rows < _E, filt * core, 0.0)
        mt_ref[...] = msg.T

    o_ref[...] = mt_ref[pl.ds(r, 1), :].reshape(_EBLK)


def _msg_flat(t, st, bn1_g, bn1_b):
    return pl.pallas_call(
        _msg_body,
        grid=(_ESTEPS, _F),
        in_specs=[
            pl.BlockSpec((_EBLK, 2 * _F), lambda i, r: (i, 0)),
            pl.BlockSpec((2, 2 * _F), lambda i, r: (0, 0)),
            pl.BlockSpec((1, 2 * _F), lambda i, r: (0, 0)),
            pl.BlockSpec((1, 2 * _F), lambda i, r: (0, 0)),
        ],
        out_specs=pl.BlockSpec((_EBLK,), lambda i, r: (r * _ESTEPS + i,)),
        out_shape=jax.ShapeDtypeStruct((_F * _EP,), _f32),
        scratch_shapes=[pltpu.VMEM((_F, _EBLK), _f32)],
    )(t, st, bn1_g.reshape(1, 2 * _F), bn1_b.reshape(1, 2 * _F))


_NSTEPS = _N // _NBLK
_UNSTEPS = _NP // _UNBLK


def _untrans_body(f_ref, s_ref, st_ref, ts_ref):
    i = pl.program_id(0)
    r = pl.program_id(1)
    ts_ref[pl.ds(r, 1), :] = f_ref[...].reshape(1, _UNBLK)

    @pl.when(r == _F - 1)
    def _():
        blk = ts_ref[...].T
        s_ref[...] = blk

        @pl.when(i == 0)
        def _():
            st_ref[...] = jnp.zeros_like(st_ref)

        st_ref[0:1, :] += jnp.sum(blk, axis=0, keepdims=True)
        st_ref[1:2, :] += jnp.sum(blk * blk, axis=0, keepdims=True)


def _untrans_stats(s_flat):
    return pl.pallas_call(
        _untrans_body,
        grid=(_UNSTEPS, _F),
        in_specs=[pl.BlockSpec((_UNBLK,), lambda i, r: (r * _UNSTEPS + i,))],
        out_specs=[
            pl.BlockSpec((_UNBLK, _F), lambda i, r: (i, 0)),
            pl.BlockSpec((2, _F), lambda i, r: (0, 0)),
        ],
        out_shape=[
            jax.ShapeDtypeStruct((_NP, _F), _f32),
            jax.ShapeDtypeStruct((2, _F), _f32),
        ],
        scratch_shapes=[pltpu.VMEM((_F, _UNBLK), _f32)],
    )(s_flat)


def _atom_body(s_ref, st_ref, g_ref, b_ref, bat_ref, wa_ref, ba_ref,
               atom_ref, gmax_ref):
    i = pl.program_id(0)
    st = st_ref[...]
    mu = st[0:1, :] * (1.0 / _N)
    var = st[1:2, :] * (1.0 / _N) - mu * mu
    atom = (s_ref[...] - mu) * lax.rsqrt(var + 1e-5) * g_ref[...] + b_ref[...]
    atom_ref[...] = atom
    a = atom * jax.nn.sigmoid(atom)
    gate = jnp.sum(a * wa_ref[...], axis=1, keepdims=True) + ba_ref[...]
    onehot = bat_ref[...] == lax.broadcasted_iota(jnp.int32, (1, _G), 1)
    masked = jnp.where(onehot, gate, -jnp.inf)

    @pl.when(i == 0)
    def _():
        gmax_ref[...] = jnp.full_like(gmax_ref, -jnp.inf)

    gmax_ref[...] = jnp.maximum(gmax_ref[...],
                                jnp.max(masked, axis=0, keepdims=True))


def _atom(s, st2, bn2_g, bn2_b, batch2, W_att, b_att):
    return pl.pallas_call(
        _atom_body,
        grid=(_NSTEPS,),
        in_specs=[
            pl.BlockSpec((_NBLK, _F), lambda i: (i, 0)),
            pl.BlockSpec((2, _F), lambda i: (0, 0)),
            pl.BlockSpec((1, _F), lambda i: (0, 0)),
            pl.BlockSpec((1, _F), lambda i: (0, 0)),
            pl.BlockSpec((_NBLK, 1), lambda i: (i, 0)),
            pl.BlockSpec((1, _F), lambda i: (0, 0)),
            pl.BlockSpec((1, 1), lambda i: (0, 0)),
        ],
        out_specs=[
            pl.BlockSpec((_NBLK, _F), lambda i: (i, 0)),
            pl.BlockSpec((1, _G), lambda i: (0, 0)),
        ],
        out_shape=[
            jax.ShapeDtypeStruct((_N, _F), _f32),
            jax.ShapeDtypeStruct((1, _G), _f32),
        ],
    )(s, st2, bn2_g.reshape(1, _F), bn2_b.reshape(1, _F), batch2,
      W_att.reshape(1, _F), b_att.reshape(1, 1))


def _pool_body(atom_ref, bat_ref, gmax_ref, wa_ref, ba_ref, wo_ref, bo_ref,
               out_ref, num_acc, den_acc):
    i = pl.program_id(0)
    atom = atom_ref[...]
    a = atom * jax.nn.sigmoid(atom)
    gate = jnp.sum(a * wa_ref[...], axis=1, keepdims=True) + ba_ref[...]
    onehot = bat_ref[...] == lax.broadcasted_iota(jnp.int32, (1, _G), 1)
    gmax_row = jnp.max(jnp.where(onehot, gmax_ref[...], -jnp.inf), axis=1,
                       keepdims=True)
    ge = jnp.exp(gate - gmax_row)
    oh_f = onehot.astype(_f32)

    @pl.when(i == 0)
    def _():
        num_acc[...] = jnp.zeros_like(num_acc)
        den_acc[...] = jnp.zeros_like(den_acc)

    dn = (((0,), (0,)), ((), ()))
    num_acc[...] += lax.dot_general(oh_f, ge * a, dn,
                                    preferred_element_type=_f32)
    den_acc[...] += lax.dot_general(oh_f, ge, dn,
                                    preferred_element_type=_f32)

    @pl.when(i == _NSTEPS - 1)
    def _():
        crys = num_acc[...] / (den_acc[...] + 1e-16)
        out_ref[...] = jnp.dot(crys, wo_ref[...],
                               preferred_element_type=_f32) + bo_ref[...]


def _pool(atom, batch2, gmax, W_att, b_att, W_out, b_out):
    return pl.pallas_call(
        _pool_body,
        grid=(_NSTEPS,),
        in_specs=[
            pl.BlockSpec((_NBLK, _F), lambda i: (i, 0)),
            pl.BlockSpec((_NBLK, 1), lambda i: (i, 0)),
            pl.BlockSpec((1, _G), lambda i: (0, 0)),
            pl.BlockSpec((1, _F), lambda i: (0, 0)),
            pl.BlockSpec((1, 1), lambda i: (0, 0)),
            pl.BlockSpec((_F, 1), lambda i: (0, 0)),
            pl.BlockSpec((1, 1), lambda i: (0, 0)),
        ],
        out_specs=pl.BlockSpec((_G, 1), lambda i: (0, 0)),
        out_shape=jax.ShapeDtypeStruct((_G, 1), _f32),
        scratch_shapes=[
            pltpu.VMEM((_G, _F), _f32),
            pltpu.VMEM((_G, 1), _f32),
        ],
    )(atom, batch2, gmax, W_att.reshape(1, _F), b_att.reshape(1, 1),
      W_out, b_out.reshape(1, 1))


def _gather_rows(table, idx):
    per_w = _EP // _NW
    mesh = plsc.VectorSubcoreMesh(core_axis_name="c", subcore_axis_name="s",
                                  num_cores=_NC, num_subcores=_NS)

    @functools.partial(
        pl.kernel,
        out_type=jax.ShapeDtypeStruct((_EP, 2 * _F), _f32),
        mesh=mesh,
        scratch_types=[
            pltpu.VMEM((_GCH,), jnp.int32),
            pltpu.VMEM((_GCH, 2 * _F), _f32),
            pltpu.SemaphoreType.DMA,
        ],
    )
    def k(table_hbm, idx_hbm, out_hbm, idx_v, rows_v, sem):
        wid = lax.axis_index("s") * _NC + lax.axis_index("c")
        base = wid * per_w

        def body(i, _):
            off = base + i * _GCH
            pltpu.sync_copy(idx_hbm.at[pl.ds(off, _GCH)], idx_v)
            pltpu.async_copy(table_hbm.at[idx_v], rows_v, sem).wait()
            pltpu.sync_copy(rows_v, out_hbm.at[pl.ds(off, _GCH)])
            return 0

        lax.fori_loop(0, per_w // _GCH, body, 0)

    return k(table, idx)


def _scatter_add_cols(msg_flat, src, zeros_col):
    mesh = plsc.VectorSubcoreMesh(core_axis_name="c", subcore_axis_name="s",
                                  num_cores=_NC, num_subcores=_NS)

    @functools.partial(
        pl.kernel,
        out_type=jax.ShapeDtypeStruct((_F * _NP,), _f32),
        mesh=mesh,
        compiler_params=pltpu.CompilerParams(needs_layout_passes=False),
        scratch_types=[
            pltpu.VMEM((_SCH,), jnp.int32),
            pltpu.VMEM((_SCH,), _f32),
            pltpu.VMEM((_SCH,), _f32),
            pltpu.VMEM((_NP,), _f32),
            pltpu.VMEM((_NP,), _f32),
        ],
    )
    def k(msg_hbm, src_hbm, zero_hbm, out_hbm, idx_v, u0_v, u1_v, acc0, acc1):
        wid = lax.axis_index("s") * _NC + lax.axis_index("c")
        r0 = 2 * wid
        pltpu.sync_copy(zero_hbm, acc0)
        pltpu.sync_copy(zero_hbm, acc1)

        def body(i, _):
            off = i * _SCH
            pltpu.sync_copy(src_hbm.at[pl.ds(off, _SCH)], idx_v)
            pltpu.sync_copy(msg_hbm.at[pl.ds(r0 * _EP + off, _SCH)], u0_v)
            pltpu.sync_copy(msg_hbm.at[pl.ds((r0 + 1) * _EP + off, _SCH)], u1_v)
            for j in range(_SCH // 16):
                sl = pl.ds(j * 16, 16)
                iv = idx_v[sl]
                plsc.addupdate_scatter(acc0, [iv], u0_v[sl])
                plsc.addupdate_scatter(acc1, [iv], u1_v[sl])
            return 0

        lax.fori_loop(0, _EP // _SCH, body, 0)
        pltpu.sync_copy(acc0, out_hbm.at[pl.ds(r0 * _NP, _NP)])
        pltpu.sync_copy(acc1, out_hbm.at[pl.ds((r0 + 1) * _NP, _NP)])

    return k(msg_flat, src, zeros_col)


def kernel(x, edge_index, edge_attr, batch, lower_f, W_emb, b_emb, W_full,
           b_full, bn1_g, bn1_b, bn2_g, bn2_b, W_att, b_att, W_out, b_out):
    src = edge_index[0]
    dst = edge_index[1]
    W_i = W_full[:_F]
    W_j = W_full[_F:2 * _F]
    W_e = W_full[2 * _F:]
    batch2 = batch.reshape(_N, 1)
    zeros_col = jnp.zeros((_NP,), _f32)
    src_p = jnp.pad(src, (0, _EP - _E))
    dst_p = jnp.pad(dst, (0, _EP - _E))
    ea_p = jnp.pad(edge_attr, ((0, _EP - _E), (0, 0)))

    hi, hj = _prep(x, lower_f, W_emb, b_emb, W_i, W_j)
    xi = _gather_rows(hi, src_p)
    xj = _gather_rows(hj, dst_p)
    t, st1 = _edge_mm(xi, xj, ea_p, W_e, b_full)
    msg_flat = _msg_flat(t, st1, bn1_g, bn1_b)
    s_flat = _scatter_add_cols(msg_flat, src_p, zeros_col)
    s_pad, st2 = _untrans_stats(s_flat)
    s = s_pad[:_N]
    atom_fea, gmax = _atom(s, st2, bn2_g, bn2_b, batch2, W_att, b_att)
    crys_fea = _pool(atom_fea, batch2, gmax, W_att, b_att, W_out, b_out)
    return (atom_fea, crys_fea)


# speedup vs baseline: 2.0765x; 2.0765x over previous
import functools

import jax
import jax.numpy as jnp
from jax import lax
from jax.experimental import pallas as pl
from jax.experimental.pallas import tpu as pltpu
from jax.experimental.pallas import tpu_sc as plsc

_N = 50000
_E = 800000
_F_IN = 128
_F = 64
_FE = 16
_G = 64

_NC = 2
_NS = 16
_NW = _NC * _NS

_EP = 819200
_NP = 51200

_NBLK = 2000
_EBLK = 4096
_UNBLK = 2048
_GCH = 800
_SCH = 6400

_f32 = jnp.float32


def _prep_body(x_ref, lf_ref, we_ref, be_ref, wi_ref, wj_ref,
               hi_ref, hj_ref):
    h = (jnp.dot(x_ref[...], we_ref[...], preferred_element_type=_f32)
         + be_ref[...] + lf_ref[...])
    hi_ref[...] = jnp.dot(h, wi_ref[...], preferred_element_type=_f32)
    hj_ref[...] = jnp.dot(h, wj_ref[...], preferred_element_type=_f32)


def _prep(x, lower_f, W_emb, b_emb, W_i, W_j):
    return pl.pallas_call(
        _prep_body,
        grid=(_N // _NBLK,),
        in_specs=[
            pl.BlockSpec((_NBLK, _F_IN), lambda i: (i, 0)),
            pl.BlockSpec((_NBLK, _F), lambda i: (i, 0)),
            pl.BlockSpec((_F_IN, _F), lambda i: (0, 0)),
            pl.BlockSpec((1, _F), lambda i: (0, 0)),
            pl.BlockSpec((_F, 2 * _F), lambda i: (0, 0)),
            pl.BlockSpec((_F, 2 * _F), lambda i: (0, 0)),
        ],
        out_specs=[
            pl.BlockSpec((_NBLK, 2 * _F), lambda i: (i, 0)),
            pl.BlockSpec((_NBLK, 2 * _F), lambda i: (i, 0)),
        ],
        out_shape=[
            jax.ShapeDtypeStruct((_N, 2 * _F), _f32),
            jax.ShapeDtypeStruct((_N, 2 * _F), _f32),
        ],
    )(x, lower_f, W_emb, b_emb.reshape(1, _F), W_i, W_j)


def _edge_mm_body(xi_ref, xj_ref, ea_ref, we_ref, b_ref, t_ref, st_ref):
    i = pl.program_id(0)
    t = (xi_ref[...] + xj_ref[...] + b_ref[...]
         + jnp.dot(ea_ref[...], we_ref[...], preferred_element_type=_f32))
    t_ref[...] = t

    @pl.when(i == 0)
    def _():
        st_ref[...] = jnp.zeros_like(st_ref)

    rows = lax.broadcasted_iota(jnp.int32, (_EBLK, 1), 0) + i * _EBLK
    tm = jnp.where(rows < _E, t, 0.0)
    st_ref[0:1, :] += jnp.sum(tm, axis=0, keepdims=True)
    st_ref[1:2, :] += jnp.sum(tm * tm, axis=0, keepdims=True)


def _edge_mm(xi, xj, ea, W_e, b_full):
    return pl.pallas_call(
        _edge_mm_body,
        grid=(_EP // _EBLK,),
        in_specs=[
            pl.BlockSpec((_EBLK, 2 * _F), lambda i: (i, 0)),
            pl.BlockSpec((_EBLK, 2 * _F), lambda i: (i, 0)),
            pl.BlockSpec((_EBLK, _FE), lambda i: (i, 0)),
            pl.BlockSpec((_FE, 2 * _F), lambda i: (0, 0)),
            pl.BlockSpec((1, 2 * _F), lambda i: (0, 0)),
        ],
        out_specs=[
            pl.BlockSpec((_EBLK, 2 * _F), lambda i: (i, 0)),
            pl.BlockSpec((2, 2 * _F), lambda i: (0, 0)),
        ],
        out_shape=[
            jax.ShapeDtypeStruct((_EP, 2 * _F), _f32),
            jax.ShapeDtypeStruct((2, 2 * _F), _f32),
        ],
    )(xi, xj, ea, W_e, b_full.reshape(1, 2 * _F))


_ESTEPS = _EP // _EBLK


def _msg_body(t_ref, st_ref, g_ref, b_ref, o_ref):
    i = pl.program_id(0)
    rows = lax.broadcasted_iota(jnp.int32, (_EBLK, 1), 0) + i * _EBLK
    st = st_ref[...]
    mu = st[0:1, :] * (1.0 / _E)
    var = st[1:2, :] * (1.0 / _E) - mu * mu
    tn = (t_ref[...] - mu) * lax.rsqrt(var + 1e-5) * g_ref[...] + b_ref[...]
    filt = jax.nn.sigmoid(tn[:, :_F])
    core = jax.nn.softplus(tn[:, _F:])
    msg = jnp.where(rows < _E, filt * core, 0.0)
    o_ref[...] = msg.T


def _msg_flat(t, st, bn1_g, bn1_b):
    mt = pl.pallas_call(
        _msg_body,
        grid=(_ESTEPS,),
        in_specs=[
            pl.BlockSpec((_EBLK, 2 * _F), lambda i: (i, 0)),
            pl.BlockSpec((2, 2 * _F), lambda i: (0, 0)),
            pl.BlockSpec((1, 2 * _F), lambda i: (0, 0)),
            pl.BlockSpec((1, 2 * _F), lambda i: (0, 0)),
        ],
        out_specs=pl.BlockSpec((_F, _EBLK), lambda i: (0, i)),
        out_shape=jax.ShapeDtypeStruct((_F, _EP), _f32),
    )(t, st, bn1_g.reshape(1, 2 * _F), bn1_b.reshape(1, 2 * _F))
    return mt.reshape(_F * _EP)


_NSTEPS = _N // _NBLK


def _untrans_body(f_ref, s_ref, st_ref):
    i = pl.program_id(0)
    blk = f_ref[...].T

    @pl.when(i == 0)
    def _():
        st_ref[...] = jnp.zeros_like(st_ref)

    s_ref[...] = blk
    st_ref[0:1, :] += jnp.sum(blk, axis=0, keepdims=True)
    st_ref[1:2, :] += jnp.sum(blk * blk, axis=0, keepdims=True)


def _untrans_stats(s_flat):
    return pl.pallas_call(
        _untrans_body,
        grid=(_NP // _UNBLK,),
        in_specs=[pl.BlockSpec((_F, _UNBLK), lambda i: (0, i))],
        out_specs=[
            pl.BlockSpec((_UNBLK, _F), lambda i: (i, 0)),
            pl.BlockSpec((2, _F), lambda i: (0, 0)),
        ],
        out_shape=[
            jax.ShapeDtypeStruct((_NP, _F), _f32),
            jax.ShapeDtypeStruct((2, _F), _f32),
        ],
    )(s_flat.reshape(_F, _NP))


def _atom_body(s_ref, st_ref, g_ref, b_ref, bat_ref, wa_ref, ba_ref,
               atom_ref, gmax_ref):
    i = pl.program_id(0)
    st = st_ref[...]
    mu = st[0:1, :] * (1.0 / _N)
    var = st[1:2, :] * (1.0 / _N) - mu * mu
    atom = (s_ref[...] - mu) * lax.rsqrt(var + 1e-5) * g_ref[...] + b_ref[...]
    atom_ref[...] = atom
    a = atom * jax.nn.sigmoid(atom)
    gate = jnp.sum(a * wa_ref[...], axis=1, keepdims=True) + ba_ref[...]
    onehot = bat_ref[...] == lax.broadcasted_iota(jnp.int32, (1, _G), 1)
    masked = jnp.where(onehot, gate, -jnp.inf)

    @pl.when(i == 0)
    def _():
        gmax_ref[...] = jnp.full_like(gmax_ref, -jnp.inf)

    gmax_ref[...] = jnp.maximum(gmax_ref[...],
                                jnp.max(masked, axis=0, keepdims=True))


def _atom(s, st2, bn2_g, bn2_b, batch2, W_att, b_att):
    return pl.pallas_call(
        _atom_body,
        grid=(_NSTEPS,),
        in_specs=[
            pl.BlockSpec((_NBLK, _F), lambda i: (i, 0)),
            pl.BlockSpec((2, _F), lambda i: (0, 0)),
            pl.BlockSpec((1, _F), lambda i: (0, 0)),
            pl.BlockSpec((1, _F), lambda i: (0, 0)),
            pl.BlockSpec((_NBLK, 1), lambda i: (i, 0)),
            pl.BlockSpec((1, _F), lambda i: (0, 0)),
            pl.BlockSpec((1, 1), lambda i: (0, 0)),
        ],
        out_specs=[
            pl.BlockSpec((_NBLK, _F), lambda i: (i, 0)),
            pl.BlockSpec((1, _G), lambda i: (0, 0)),
        ],
        out_shape=[
            jax.ShapeDtypeStruct((_N, _F), _f32),
            jax.ShapeDtypeStruct((1, _G), _f32),
        ],
    )(s, st2, bn2_g.reshape(1, _F), bn2_b.reshape(1, _F), batch2,
      W_att.reshape(1, _F), b_att.reshape(1, 1))


def _pool_body(atom_ref, bat_ref, gmax_ref, wa_ref, ba_ref, wo_ref, bo_ref,
               out_ref, num_acc, den_acc):
    i = pl.program_id(0)
    atom = atom_ref[...]
    a = atom * jax.nn.sigmoid(atom)
    gate = jnp.sum(a * wa_ref[...], axis=1, keepdims=True) + ba_ref[...]
    onehot = bat_ref[...] == lax.broadcasted_iota(jnp.int32, (1, _G), 1)
    gmax_row = jnp.max(jnp.where(onehot, gmax_ref[...], -jnp.inf), axis=1,
                       keepdims=True)
    ge = jnp.exp(gate - gmax_row)
    oh_f = onehot.astype(_f32)

    @pl.when(i == 0)
    def _():
        num_acc[...] = jnp.zeros_like(num_acc)
        den_acc[...] = jnp.zeros_like(den_acc)

    dn = (((0,), (0,)), ((), ()))
    num_acc[...] += lax.dot_general(oh_f, ge * a, dn,
                                    preferred_element_type=_f32)
    den_acc[...] += lax.dot_general(oh_f, ge, dn,
                                    preferred_element_type=_f32)

    @pl.when(i == _NSTEPS - 1)
    def _():
        crys = num_acc[...] / (den_acc[...] + 1e-16)
        out_ref[...] = jnp.dot(crys, wo_ref[...],
                               preferred_element_type=_f32) + bo_ref[...]


def _pool(atom, batch2, gmax, W_att, b_att, W_out, b_out):
    return pl.pallas_call(
        _pool_body,
        grid=(_NSTEPS,),
        in_specs=[
            pl.BlockSpec((_NBLK, _F), lambda i: (i, 0)),
            pl.BlockSpec((_NBLK, 1), lambda i: (i, 0)),
            pl.BlockSpec((1, _G), lambda i: (0, 0)),
            pl.BlockSpec((1, _F), lambda i: (0, 0)),
            pl.BlockSpec((1, 1), lambda i: (0, 0)),
            pl.BlockSpec((_F, 1), lambda i: (0, 0)),
            pl.BlockSpec((1, 1), lambda i: (0, 0)),
        ],
        out_specs=pl.BlockSpec((_G, 1), lambda i: (0, 0)),
        out_shape=jax.ShapeDtypeStruct((_G, 1), _f32),
        scratch_shapes=[
            pltpu.VMEM((_G, _F), _f32),
            pltpu.VMEM((_G, 1), _f32),
        ],
    )(atom, batch2, gmax, W_att.reshape(1, _F), b_att.reshape(1, 1),
      W_out, b_out.reshape(1, 1))


def _gather_rows(table, idx):
    per_w = _EP // _NW
    mesh = plsc.VectorSubcoreMesh(core_axis_name="c", subcore_axis_name="s",
                                  num_cores=_NC, num_subcores=_NS)

    @functools.partial(
        pl.kernel,
        out_type=jax.ShapeDtypeStruct((_EP, 2 * _F), _f32),
        mesh=mesh,
        scratch_types=[
            pltpu.VMEM((_GCH,), jnp.int32),
            pltpu.VMEM((_GCH, 2 * _F), _f32),
            pltpu.SemaphoreType.DMA,
        ],
    )
    def k(table_hbm, idx_hbm, out_hbm, idx_v, rows_v, sem):
        wid = lax.axis_index("s") * _NC + lax.axis_index("c")
        base = wid * per_w

        def body(i, _):
            off = base + i * _GCH
            pltpu.sync_copy(idx_hbm.at[pl.ds(off, _GCH)], idx_v)
            pltpu.async_copy(table_hbm.at[idx_v], rows_v, sem).wait()
            pltpu.sync_copy(rows_v, out_hbm.at[pl.ds(off, _GCH)])
            return 0

        lax.fori_loop(0, per_w // _GCH, body, 0)

    return k(table, idx)


def _scatter_add_cols(msg_flat, src, zeros_col):
    mesh = plsc.VectorSubcoreMesh(core_axis_name="c", subcore_axis_name="s",
                                  num_cores=_NC, num_subcores=_NS)

    @functools.partial(
        pl.kernel,
        out_type=jax.ShapeDtypeStruct((_F * _NP,), _f32),
        mesh=mesh,
        compiler_params=pltpu.CompilerParams(needs_layout_passes=False),
        scratch_types=[
            pltpu.VMEM((_SCH,), jnp.int32),
            pltpu.VMEM((_SCH,), _f32),
            pltpu.VMEM((_SCH,), _f32),
            pltpu.VMEM((_NP,), _f32),
            pltpu.VMEM((_NP,), _f32),
        ],
    )
    def k(msg_hbm, src_hbm, zero_hbm, out_hbm, idx_v, u0_v, u1_v, acc0, acc1):
        wid = lax.axis_index("s") * _NC + lax.axis_index("c")
        r0 = 2 * wid
        pltpu.sync_copy(zero_hbm, acc0)
        pltpu.sync_copy(zero_hbm, acc1)

        def body(i, _):
            off = i * _SCH
            pltpu.sync_copy(src_hbm.at[pl.ds(off, _SCH)], idx_v)
            pltpu.sync_copy(msg_hbm.at[pl.ds(r0 * _EP + off, _SCH)], u0_v)
            pltpu.sync_copy(msg_hbm.at[pl.ds((r0 + 1) * _EP + off, _SCH)], u1_v)
            for j in range(_SCH // 16):
                sl = pl.ds(j * 16, 16)
                iv = idx_v[sl]
                plsc.addupdate_scatter(acc0, [iv], u0_v[sl])
                plsc.addupdate_scatter(acc1, [iv], u1_v[sl])
            return 0

        lax.fori_loop(0, _EP // _SCH, body, 0)
        pltpu.sync_copy(acc0, out_hbm.at[pl.ds(r0 * _NP, _NP)])
        pltpu.sync_copy(acc1, out_hbm.at[pl.ds((r0 + 1) * _NP, _NP)])

    return k(msg_flat, src, zeros_col)


def kernel(x, edge_index, edge_attr, batch, lower_f, W_emb, b_emb, W_full,
           b_full, bn1_g, bn1_b, bn2_g, bn2_b, W_att, b_att, W_out, b_out):
    src = edge_index[0]
    dst = edge_index[1]
    W_i = W_full[:_F]
    W_j = W_full[_F:2 * _F]
    W_e = W_full[2 * _F:]
    batch2 = batch.reshape(_N, 1)
    zeros_col = jnp.zeros((_NP,), _f32)
    src_p = jnp.pad(src, (0, _EP - _E))
    dst_p = jnp.pad(dst, (0, _EP - _E))
    ea_p = jnp.pad(edge_attr, ((0, _EP - _E), (0, 0)))

    hi, hj = _prep(x, lower_f, W_emb, b_emb, W_i, W_j)
    xi = _gather_rows(hi, src_p)
    xj = _gather_rows(hj, dst_p)
    t, st1 = _edge_mm(xi, xj, ea_p, W_e, b_full)
    msg_flat = _msg_flat(t, st1, bn1_g, bn1_b)
    s_flat = _scatter_add_cols(msg_flat, src_p, zeros_col)
    s_pad, st2 = _untrans_stats(s_flat)
    s = s_pad[:_N]
    atom_fea, gmax = _atom(s, st2, bn2_g, bn2_b, batch2, W_att, b_att)
    crys_fea = _pool(atom_fea, batch2, gmax, W_att, b_att, W_out, b_out)
    return (atom_fea, crys_fea)
